# baseline (speedup 1.0000x reference)
import functools

import jax
import jax.numpy as jnp
from jax.experimental import pallas as pl
from jax.experimental.pallas import tpu as pltpu

B_, S_, D_ = 4, 8192, 768
E_, R_, K_ = 8, 4, 2
W_ = E_ * R_ + 8


def _fused_body(x_ref, wb_ref, rw_ref, a_ref, bm_ref, out_ref):
    xt = x_ref[...]
    xb = xt.astype(jnp.bfloat16)
    base = jnp.dot(xb, wb_ref[...], preferred_element_type=jnp.float32)
    logits_t = jax.lax.dot_general(
        rw_ref[...], xt, (((1,), (1,)), ((), ())),
        preferred_element_type=jnp.float32)
    h_t = jax.lax.dot_general(
        a_ref[...], xb, (((1,), (1,)), ((), ())),
        preferred_element_type=jnp.float32)

    mx = jnp.max(logits_t, axis=0, keepdims=True)
    ex = jnp.exp(logits_t - mx)
    z = jnp.sum(ex, axis=0, keepdims=True)

    idx = jax.lax.broadcasted_iota(jnp.int32, ex.shape, 0)
    m1 = jnp.max(ex, axis=0, keepdims=True)
    i1 = jnp.min(jnp.where(ex == m1, idx, E_), axis=0, keepdims=True)
    pm = jnp.where(idx == i1, -jnp.inf, ex)
    m2 = jnp.max(pm, axis=0, keepdims=True)
    i2 = jnp.min(jnp.where(pm == m2, idx, E_), axis=0, keepdims=True)
    denom = m1 + m2 + 1e-6 * z
    g_t = (jnp.where(idx == i1, m1, 0.0) + jnp.where(idx == i2, m2, 0.0)) / denom

    one0 = (idx == 0).astype(jnp.float32)
    wh_t = jnp.concatenate(
        [(h_t * jnp.concatenate([g_t] * R_, axis=0)), one0],
        axis=0).astype(jnp.bfloat16)
    y = jax.lax.dot_general(
        wh_t, bm_ref[...], (((0,), (0,)), ((), ())),
        preferred_element_type=jnp.float32)
    out_ref[...] = base + y


@functools.partial(jax.jit, static_argnames=("tile_m",))
def _run(x2d, wbT, rwT, aT, bmT, tile_m=2048):
    m = x2d.shape[0]
    grid = (m // tile_m,)
    return pl.pallas_call(
        _fused_body,
        grid=grid,
        in_specs=[
            pl.BlockSpec((tile_m, D_), lambda i: (i, 0)),
            pl.BlockSpec((D_, D_), lambda i: (0, 0)),
            pl.BlockSpec((E_, D_), lambda i: (0, 0)),
            pl.BlockSpec((E_ * R_, D_), lambda i: (0, 0)),
            pl.BlockSpec((W_, D_), lambda i: (0, 0)),
        ],
        out_specs=pl.BlockSpec((tile_m, D_), lambda i: (i, 0)),
        out_shape=jax.ShapeDtypeStruct((m, D_), jnp.float32),
        compiler_params=pltpu.CompilerParams(
            dimension_semantics=("parallel",)),
    )(x2d, wbT, rwT, aT, bmT)


def kernel(x, base_W, base_b, router_W, A, Bm):
    b, s, d = x.shape
    x2d = x.reshape(b * s, d)
    wbT = base_W.T.astype(jnp.bfloat16)
    rwT = router_W
    aT = jnp.transpose(A, (1, 0, 2)).reshape(R_ * E_, D_).astype(jnp.bfloat16)
    bmT = jnp.concatenate([
        jnp.transpose(Bm, (2, 0, 1)).reshape(R_ * E_, D_),
        base_b.reshape(1, D_),
        jnp.zeros((W_ - E_ * R_ - 1, D_), jnp.float32),
    ], axis=0).astype(jnp.bfloat16)
    out = _run(x2d, wbT, rwT, aT, bmT)
    return out.reshape(b, s, d)


# speedup vs baseline: 1.0009x; 1.0009x over previous
import functools

import jax
import jax.numpy as jnp
from jax.experimental import pallas as pl
from jax.experimental.pallas import tpu as pltpu

B_, S_, D_ = 4, 8192, 768
E_, R_, K_ = 8, 4, 2
W_ = E_ * R_ + 8


def _fused_body(x_ref, wb_ref, rw_ref, a_ref, bm_ref, out_ref):
    xt = x_ref[...]
    xb = xt.astype(jnp.bfloat16)
    base = jnp.dot(xb, wb_ref[...], preferred_element_type=jnp.float32)
    logits_t = jax.lax.dot_general(
        rw_ref[...], xt, (((1,), (1,)), ((), ())),
        preferred_element_type=jnp.float32)
    h_t = jax.lax.dot_general(
        a_ref[...], xb, (((1,), (1,)), ((), ())),
        preferred_element_type=jnp.float32)

    mx = jnp.max(logits_t, axis=0, keepdims=True)
    ex = jnp.exp(logits_t - mx)
    z = jnp.sum(ex, axis=0, keepdims=True)

    idx = jax.lax.broadcasted_iota(jnp.int32, ex.shape, 0)
    m1 = jnp.max(ex, axis=0, keepdims=True)
    i1 = jnp.min(jnp.where(ex == m1, idx, E_), axis=0, keepdims=True)
    pm = jnp.where(idx == i1, -jnp.inf, ex)
    m2 = jnp.max(pm, axis=0, keepdims=True)
    i2 = jnp.min(jnp.where(pm == m2, idx, E_), axis=0, keepdims=True)
    denom = m1 + m2 + 1e-6 * z
    g_t = (jnp.where(idx == i1, m1, 0.0) + jnp.where(idx == i2, m2, 0.0)) / denom

    one0 = (idx == 0).astype(jnp.float32)
    wh_t = jnp.concatenate(
        [(h_t * jnp.concatenate([g_t] * R_, axis=0)), one0],
        axis=0).astype(jnp.bfloat16)
    y = jnp.dot(jnp.transpose(wh_t), bm_ref[...],
                preferred_element_type=jnp.float32)
    out_ref[...] = base + y


@functools.partial(jax.jit, static_argnames=("tile_m",))
def _run(x2d, wbT, rwT, aT, bmT, tile_m=2048):
    m = x2d.shape[0]
    grid = (m // tile_m,)
    return pl.pallas_call(
        _fused_body,
        grid=grid,
        in_specs=[
            pl.BlockSpec((tile_m, D_), lambda i: (i, 0)),
            pl.BlockSpec((D_, D_), lambda i: (0, 0)),
            pl.BlockSpec((E_, D_), lambda i: (0, 0)),
            pl.BlockSpec((E_ * R_, D_), lambda i: (0, 0)),
            pl.BlockSpec((W_, D_), lambda i: (0, 0)),
        ],
        out_specs=pl.BlockSpec((tile_m, D_), lambda i: (i, 0)),
        out_shape=jax.ShapeDtypeStruct((m, D_), jnp.float32),
        compiler_params=pltpu.CompilerParams(
            dimension_semantics=("parallel",)),
    )(x2d, wbT, rwT, aT, bmT)


def kernel(x, base_W, base_b, router_W, A, Bm):
    b, s, d = x.shape
    x2d = x.reshape(b * s, d)
    wbT = base_W.T.astype(jnp.bfloat16)
    rwT = router_W
    aT = jnp.transpose(A, (1, 0, 2)).reshape(R_ * E_, D_).astype(jnp.bfloat16)
    bmT = jnp.concatenate([
        jnp.transpose(Bm, (2, 0, 1)).reshape(R_ * E_, D_),
        base_b.reshape(1, D_),
        jnp.zeros((W_ - E_ * R_ - 1, D_), jnp.float32),
    ], axis=0).astype(jnp.bfloat16)
    out = _run(x2d, wbT, rwT, aT, bmT)
    return out.reshape(b, s, d)
